# baseline (speedup 1.0000x reference)
import functools

import jax
import jax.numpy as jnp
from jax import lax
from jax.experimental import pallas as pl
from jax.experimental.pallas import tpu as pltpu
from jax.experimental.pallas import tpu_sc as plsc

QN = 1024
KN = 65536
DN = 512
TK = 8
CHUNK = 128
NCHUNK = KN // CHUNK
BK = 2048
NBLK = KN // BK
CPB = BK // CHUNK
NCAND = 2 * NCHUNK


def _matmul_top2_body(q_ref, f_ref, cval_ref, cidx_ref):
    k = pl.program_id(0)
    s = lax.dot_general(
        q_ref[...].astype(jnp.bfloat16), f_ref[...].astype(jnp.bfloat16),
        (((1,), (1,)), ((), ())),
        preferred_element_type=jnp.float32)
    s3 = s.reshape(QN, CPB, CHUNK)
    lane = lax.broadcasted_iota(jnp.int32, (QN, CPB, CHUNK), 2)
    neg = jnp.float32(-jnp.inf)
    m1 = jnp.max(s3, axis=2)
    k1 = jnp.min(jnp.where(s3 == m1[:, :, None], lane, CHUNK), axis=2)
    s3m = jnp.where(lane == k1[:, :, None], neg, s3)
    m2 = jnp.max(s3m, axis=2)
    k2 = jnp.min(jnp.where(s3m == m2[:, :, None], lane, CHUNK), axis=2)
    base = k * BK + lax.broadcasted_iota(jnp.int32, (QN, CPB), 1) * CHUNK
    cval_ref[...] = jnp.concatenate([m1, m2], axis=1)[None]
    cidx_ref[...] = jnp.concatenate([base + k1, base + k2], axis=1)[None]


def _matmul_top2(queries, features):
    return pl.pallas_call(
        _matmul_top2_body,
        grid=(NBLK,),
        in_specs=[
            pl.BlockSpec((QN, DN), lambda k: (0, 0)),
            pl.BlockSpec((BK, DN), lambda k: (k, 0)),
        ],
        out_specs=[
            pl.BlockSpec((1, QN, 2 * CPB), lambda k: (k, 0, 0)),
            pl.BlockSpec((1, QN, 2 * CPB), lambda k: (k, 0, 0)),
        ],
        out_shape=[
            jax.ShapeDtypeStruct((NBLK, QN, 2 * CPB), jnp.float32),
            jax.ShapeDtypeStruct((NBLK, QN, 2 * CPB), jnp.int32),
        ],
    )(queries, features)


def _final_topk_body(cval_ref, cidx_ref, out_ref):
    vals = cval_ref[...]
    gidx = cidx_ref[...]
    lane = lax.broadcasted_iota(jnp.int32, (QN, NCAND), 1)
    neg = jnp.float32(-jnp.inf)
    outs = []
    for _ in range(TK):
        m = jnp.max(vals, axis=1, keepdims=True)
        am = jnp.min(jnp.where(vals == m, lane, NCAND), axis=1, keepdims=True)
        sel = lane == am
        outs.append(jnp.max(jnp.where(sel, gidx, -1), axis=1, keepdims=True))
        vals = jnp.where(sel, neg, vals)
    out_ref[...] = jnp.concatenate(outs, axis=1)


def _final_topk(cval, cidx):
    return pl.pallas_call(
        _final_topk_body,
        out_shape=jax.ShapeDtypeStruct((QN, TK), jnp.int32),
    )(cval, cidx)


_NC = 2
_NS = 16
_NW = _NC * _NS


def _sc_gather_feats(features, idx2):
    mesh = plsc.VectorSubcoreMesh(core_axis_name="c", subcore_axis_name="s")

    @functools.partial(
        pl.kernel, mesh=mesh,
        out_type=jax.ShapeDtypeStruct((QN * TK, DN), jnp.float32),
        scratch_types=[
            pltpu.VMEM((2, 128), jnp.int32),
            pltpu.VMEM((128, DN), jnp.float32),
            pltpu.SemaphoreType.DMA,
        ],
    )
    def k(f_hbm, idx_hbm, out_hbm, idx_v, rows_v, sem):
        wid = lax.axis_index("s") * _NC + lax.axis_index("c")
        pltpu.sync_copy(idx_hbm.at[pl.ds(wid * 2, 2)], idx_v)
        for j in range(2):
            pltpu.async_copy(f_hbm.at[idx_v.at[j]], rows_v, sem).wait()
            pltpu.sync_copy(rows_v,
                            out_hbm.at[pl.ds(wid * 256 + j * 128, 128)])

    return k(features, idx2)


def kernel(queries, features):
    cval3, cidx3 = _matmul_top2(queries, features)
    cval = jnp.transpose(cval3, (1, 0, 2)).reshape(QN, NCAND)
    cidx = jnp.transpose(cidx3, (1, 0, 2)).reshape(QN, NCAND)
    fidx = _final_topk(cval, cidx)
    sel = _sc_gather_feats(features, fidx.reshape(64, 128))
    subgraph_feats = jnp.concatenate([queries, sel], axis=0)
    src = jnp.repeat(jnp.arange(QN, dtype=jnp.int64), TK)
    dst = QN + jnp.tile(jnp.arange(TK, dtype=jnp.int64), QN)
    edge_index = jnp.stack([src, dst], axis=0)
    return subgraph_feats, edge_index


# speedup vs baseline: 1.2734x; 1.2734x over previous
import functools

import jax
import jax.numpy as jnp
from jax import lax
from jax.experimental import pallas as pl
from jax.experimental.pallas import tpu as pltpu
from jax.experimental.pallas import tpu_sc as plsc

QN = 1024
KN = 65536
DN = 512
TK = 8
CHUNK = 128
NCHUNK = KN // CHUNK
BK = 2048
NBLK = KN // BK
CPB = BK // CHUNK
NCAND = 2 * NCHUNK


def _lane_of(onehot, lane_mat):
    prod = lax.dot_general(onehot, lane_mat, (((1,), (0,)), ((), ())),
                           preferred_element_type=jnp.float32)
    lanes = prod[:, 0:1].reshape(QN, CPB).astype(jnp.int32)
    return jnp.clip(lanes, 0, CHUNK - 1)


def _matmul_top2_body(q_ref, f_ref, cval_ref, cidx_ref):
    k = pl.program_id(0)
    s = lax.dot_general(
        q_ref[...].astype(jnp.bfloat16), f_ref[...].astype(jnp.bfloat16),
        (((1,), (1,)), ((), ())),
        preferred_element_type=jnp.float32)
    s3 = s.reshape(QN, CPB, CHUNK)
    neg = jnp.float32(-jnp.inf)
    one = jnp.float32(1.0)
    zero = jnp.float32(0.0)
    lane_mat = lax.broadcasted_iota(
        jnp.int32, (CHUNK, CHUNK), 0).astype(jnp.float32)
    m1 = jnp.max(s3, axis=2)
    eq1 = s3 == m1[:, :, None]
    oh1 = jnp.where(eq1, one, zero).reshape(QN * CPB, CHUNK)
    s3m = jnp.where(eq1, neg, s3)
    m2 = jnp.max(s3m, axis=2)
    oh2 = jnp.where(s3m == m2[:, :, None], one, zero).reshape(QN * CPB, CHUNK)
    k1 = _lane_of(oh1, lane_mat)
    k2 = _lane_of(oh2, lane_mat)
    base = k * BK + lax.broadcasted_iota(jnp.int32, (QN, CPB), 1) * CHUNK
    cval_ref[...] = jnp.concatenate([m1, m2], axis=1)[None]
    cidx_ref[...] = jnp.concatenate([base + k1, base + k2], axis=1)[None]


def _matmul_top2(queries, features):
    return pl.pallas_call(
        _matmul_top2_body,
        grid=(NBLK,),
        in_specs=[
            pl.BlockSpec((QN, DN), lambda k: (0, 0)),
            pl.BlockSpec((BK, DN), lambda k: (k, 0)),
        ],
        out_specs=[
            pl.BlockSpec((1, QN, 2 * CPB), lambda k: (k, 0, 0)),
            pl.BlockSpec((1, QN, 2 * CPB), lambda k: (k, 0, 0)),
        ],
        out_shape=[
            jax.ShapeDtypeStruct((NBLK, QN, 2 * CPB), jnp.float32),
            jax.ShapeDtypeStruct((NBLK, QN, 2 * CPB), jnp.int32),
        ],
    )(queries, features)


def _final_topk_body(cval_ref, cidx_ref, out_ref):
    vals = cval_ref[...]
    gidx = cidx_ref[...]
    lane = lax.broadcasted_iota(jnp.int32, (QN, NCAND), 1)
    neg = jnp.float32(-jnp.inf)
    outs = []
    for _ in range(TK):
        m = jnp.max(vals, axis=1, keepdims=True)
        am = jnp.min(jnp.where(vals == m, lane, NCAND), axis=1, keepdims=True)
        sel = lane == am
        outs.append(jnp.max(jnp.where(sel, gidx, -1), axis=1, keepdims=True))
        vals = jnp.where(sel, neg, vals)
    out_ref[...] = jnp.concatenate(outs, axis=1)


def _final_topk(cval, cidx):
    return pl.pallas_call(
        _final_topk_body,
        out_shape=jax.ShapeDtypeStruct((QN, TK), jnp.int32),
    )(cval, cidx)


_NC = 2
_NS = 16
_NW = _NC * _NS


def _sc_gather_feats(features, idx2):
    mesh = plsc.VectorSubcoreMesh(core_axis_name="c", subcore_axis_name="s")

    @functools.partial(
        pl.kernel, mesh=mesh,
        out_type=jax.ShapeDtypeStruct((QN * TK, DN), jnp.float32),
        scratch_types=[
            pltpu.VMEM((2, 128), jnp.int32),
            pltpu.VMEM((128, DN), jnp.float32),
            pltpu.SemaphoreType.DMA,
        ],
    )
    def k(f_hbm, idx_hbm, out_hbm, idx_v, rows_v, sem):
        wid = lax.axis_index("s") * _NC + lax.axis_index("c")
        pltpu.sync_copy(idx_hbm.at[pl.ds(wid * 2, 2)], idx_v)
        for j in range(2):
            pltpu.async_copy(f_hbm.at[idx_v.at[j]], rows_v, sem).wait()
            pltpu.sync_copy(rows_v,
                            out_hbm.at[pl.ds(wid * 256 + j * 128, 128)])

    return k(features, idx2)


def kernel(queries, features):
    cval3, cidx3 = _matmul_top2(queries, features)
    cval = jnp.transpose(cval3, (1, 0, 2)).reshape(QN, NCAND)
    cidx = jnp.transpose(cidx3, (1, 0, 2)).reshape(QN, NCAND)
    fidx = _final_topk(cval, cidx)
    sel = _sc_gather_feats(features, fidx.reshape(64, 128))
    subgraph_feats = jnp.concatenate([queries, sel], axis=0)
    src = jnp.repeat(jnp.arange(QN, dtype=jnp.int64), TK)
    dst = QN + jnp.tile(jnp.arange(TK, dtype=jnp.int64), QN)
    edge_index = jnp.stack([src, dst], axis=0)
    return subgraph_feats, edge_index


# speedup vs baseline: 2.3410x; 1.8384x over previous
import functools

import jax
import jax.numpy as jnp
from jax import lax
from jax.experimental import pallas as pl
from jax.experimental.pallas import tpu as pltpu
from jax.experimental.pallas import tpu_sc as plsc

QN = 1024
KN = 65536
DN = 512
TK = 8
CHUNK = 128
BK = 4096
NBLK = KN // BK
CPB = BK // CHUNK
NCAND = NBLK * 2 * CHUNK


def _matmul_top2_body(q_ref, f_ref, cval_ref, cidx_ref):
    k = pl.program_id(0)
    s = lax.dot_general(
        q_ref[...].astype(jnp.bfloat16), f_ref[...].astype(jnp.bfloat16),
        (((1,), (1,)), ((), ())),
        preferred_element_type=jnp.float32)
    neg = jnp.float32(-jnp.inf)
    sl = [s[:, p * CHUNK:(p + 1) * CHUNK] for p in range(CPB)]
    m1 = sl[0]
    m2 = jnp.full((QN, CHUNK), neg)
    for p in range(1, CPB):
        x = sl[p]
        t = jnp.minimum(m1, x)
        m1 = jnp.maximum(m1, x)
        m2 = jnp.maximum(m2, t)
    p1 = jnp.zeros((QN, CHUNK), jnp.int32)
    for p in reversed(range(CPB)):
        p1 = jnp.where(sl[p] == m1, jnp.int32(p), p1)
    p2 = jnp.zeros((QN, CHUNK), jnp.int32)
    for p in reversed(range(CPB)):
        p2 = jnp.where((sl[p] == m2) & (p1 != jnp.int32(p)), jnp.int32(p), p2)
    lanes = lax.broadcasted_iota(jnp.int32, (QN, CHUNK), 1) + k * BK
    cval_ref[...] = jnp.concatenate([m1, m2], axis=1)[None]
    cidx_ref[...] = jnp.concatenate(
        [lanes + p1 * CHUNK, lanes + p2 * CHUNK], axis=1)[None]


def _matmul_top2(queries, features):
    return pl.pallas_call(
        _matmul_top2_body,
        grid=(NBLK,),
        in_specs=[
            pl.BlockSpec((QN, DN), lambda k: (0, 0)),
            pl.BlockSpec((BK, DN), lambda k: (k, 0)),
        ],
        out_specs=[
            pl.BlockSpec((1, QN, 2 * CHUNK), lambda k: (k, 0, 0)),
            pl.BlockSpec((1, QN, 2 * CHUNK), lambda k: (k, 0, 0)),
        ],
        out_shape=[
            jax.ShapeDtypeStruct((NBLK, QN, 2 * CHUNK), jnp.float32),
            jax.ShapeDtypeStruct((NBLK, QN, 2 * CHUNK), jnp.int32),
        ],
    )(queries, features)


def _final_topk_body(cval_ref, cidx_ref, out_ref):
    vals = cval_ref[...]
    gidx = cidx_ref[...]
    lane = lax.broadcasted_iota(jnp.int32, (QN, NCAND), 1)
    neg = jnp.float32(-jnp.inf)
    outs = []
    for _ in range(TK):
        m = jnp.max(vals, axis=1, keepdims=True)
        am = jnp.min(jnp.where(vals == m, lane, NCAND), axis=1, keepdims=True)
        sel = lane == am
        outs.append(jnp.max(jnp.where(sel, gidx, -1), axis=1, keepdims=True))
        vals = jnp.where(sel, neg, vals)
    out_ref[...] = jnp.concatenate(outs, axis=1)


def _final_topk(cval, cidx):
    return pl.pallas_call(
        _final_topk_body,
        out_shape=jax.ShapeDtypeStruct((QN, TK), jnp.int32),
    )(cval, cidx)


_NC = 2
_NS = 16
_NW = _NC * _NS


def _sc_gather_feats(features, idx2):
    mesh = plsc.VectorSubcoreMesh(core_axis_name="c", subcore_axis_name="s")

    @functools.partial(
        pl.kernel, mesh=mesh,
        out_type=jax.ShapeDtypeStruct((QN * TK, DN), jnp.float32),
        scratch_types=[
            pltpu.VMEM((2, 128), jnp.int32),
            pltpu.VMEM((128, DN), jnp.float32),
            pltpu.SemaphoreType.DMA,
        ],
    )
    def k(f_hbm, idx_hbm, out_hbm, idx_v, rows_v, sem):
        wid = lax.axis_index("s") * _NC + lax.axis_index("c")
        pltpu.sync_copy(idx_hbm.at[pl.ds(wid * 2, 2)], idx_v)
        for j in range(2):
            pltpu.async_copy(f_hbm.at[idx_v.at[j]], rows_v, sem).wait()
            pltpu.sync_copy(rows_v,
                            out_hbm.at[pl.ds(wid * 256 + j * 128, 128)])

    return k(features, idx2)


def kernel(queries, features):
    cval3, cidx3 = _matmul_top2(queries, features)
    cval = jnp.transpose(cval3, (1, 0, 2)).reshape(QN, NCAND)
    cidx = jnp.transpose(cidx3, (1, 0, 2)).reshape(QN, NCAND)
    fidx = _final_topk(cval, cidx)
    sel = _sc_gather_feats(features, fidx.reshape(64, 128))
    subgraph_feats = jnp.concatenate([queries, sel], axis=0)
    src = jnp.repeat(jnp.arange(QN, dtype=jnp.int64), TK)
    dst = QN + jnp.tile(jnp.arange(TK, dtype=jnp.int64), QN)
    edge_index = jnp.stack([src, dst], axis=0)
    return subgraph_feats, edge_index


# speedup vs baseline: 3.0724x; 1.3124x over previous
import functools

import jax
import jax.numpy as jnp
from jax import lax
from jax.experimental import pallas as pl
from jax.experimental.pallas import tpu as pltpu
from jax.experimental.pallas import tpu_sc as plsc

QN = 1024
KN = 65536
DN = 512
TK = 8
CHUNK = 128
BK = 4096
NBLK = KN // BK
CPB = BK // CHUNK
NCAND = NBLK * 2 * CHUNK


def _matmul_top2_body(q_ref, f_ref, cval_ref, cidx_ref):
    k = pl.program_id(0)
    s = lax.dot_general(
        q_ref[...].astype(jnp.bfloat16), f_ref[...].astype(jnp.bfloat16),
        (((1,), (1,)), ((), ())),
        preferred_element_type=jnp.float32)
    neg = jnp.float32(-jnp.inf)
    sl = [s[:, p * CHUNK:(p + 1) * CHUNK] for p in range(CPB)]
    m1 = sl[0]
    m2 = jnp.full((QN, CHUNK), neg)
    for p in range(1, CPB):
        x = sl[p]
        t = jnp.minimum(m1, x)
        m1 = jnp.maximum(m1, x)
        m2 = jnp.maximum(m2, t)
    p1 = jnp.zeros((QN, CHUNK), jnp.int32)
    for p in reversed(range(CPB)):
        p1 = jnp.where(sl[p] == m1, jnp.int32(p), p1)
    p2 = jnp.zeros((QN, CHUNK), jnp.int32)
    for p in reversed(range(CPB)):
        p2 = jnp.where((sl[p] == m2) & (p1 != jnp.int32(p)), jnp.int32(p), p2)
    lanes = lax.broadcasted_iota(jnp.int32, (QN, CHUNK), 1) + k * BK
    cval_ref[...] = jnp.concatenate([m1, m2], axis=1)[None]
    cidx_ref[...] = jnp.concatenate(
        [lanes + p1 * CHUNK, lanes + p2 * CHUNK], axis=1)[None]


def _matmul_top2(queries, features):
    return pl.pallas_call(
        _matmul_top2_body,
        grid=(NBLK,),
        in_specs=[
            pl.BlockSpec((QN, DN), lambda k: (0, 0)),
            pl.BlockSpec((BK, DN), lambda k: (k, 0)),
        ],
        out_specs=[
            pl.BlockSpec((1, QN, 2 * CHUNK), lambda k: (k, 0, 0)),
            pl.BlockSpec((1, QN, 2 * CHUNK), lambda k: (k, 0, 0)),
        ],
        out_shape=[
            jax.ShapeDtypeStruct((NBLK, QN, 2 * CHUNK), jnp.float32),
            jax.ShapeDtypeStruct((NBLK, QN, 2 * CHUNK), jnp.int32),
        ],
    )(queries, features)


NC4 = 4 * CHUNK


TQ = 128


def _final_topk_body(cval_ref, cidx_ref, out_ref):
    neg = jnp.float32(-jnp.inf)
    v = [jnp.full((TQ, CHUNK), neg) for _ in range(4)]
    gi = [jnp.zeros((TQ, CHUNK), jnp.int32) for _ in range(4)]
    for b in range(NBLK):
        for r in range(2):
            x = cval_ref[b, :, r * CHUNK:(r + 1) * CHUNK]
            xi = cidx_ref[b, :, r * CHUNK:(r + 1) * CHUNK]
            c = [x > v[j] for j in range(4)]
            nv1 = jnp.maximum(v[0], x)
            ni1 = jnp.where(c[0], xi, gi[0])
            nv2 = jnp.where(c[0], v[0], jnp.where(c[1], x, v[1]))
            ni2 = jnp.where(c[0], gi[0], jnp.where(c[1], xi, gi[1]))
            nv3 = jnp.where(c[1], v[1], jnp.where(c[2], x, v[2]))
            ni3 = jnp.where(c[1], gi[1], jnp.where(c[2], xi, gi[2]))
            nv4 = jnp.where(c[2], v[2], jnp.where(c[3], x, v[3]))
            ni4 = jnp.where(c[2], gi[2], jnp.where(c[3], xi, gi[3]))
            v = [nv1, nv2, nv3, nv4]
            gi = [ni1, ni2, ni3, ni4]
    vals = jnp.concatenate(v, axis=1)
    gidx = jnp.concatenate(gi, axis=1)
    lane = lax.broadcasted_iota(jnp.int32, (TQ, NC4), 1)
    outs = []
    for _ in range(TK):
        m = jnp.max(vals, axis=1, keepdims=True)
        am = jnp.min(jnp.where(vals == m, lane, NC4), axis=1, keepdims=True)
        sel = lane == am
        outs.append(jnp.max(jnp.where(sel, gidx, -1), axis=1, keepdims=True))
        vals = jnp.where(sel, neg, vals)
    out_ref[...] = jnp.concatenate(outs, axis=1)


def _final_topk(cval3, cidx3):
    return pl.pallas_call(
        _final_topk_body,
        grid=(QN // TQ,),
        in_specs=[
            pl.BlockSpec((NBLK, TQ, 2 * CHUNK), lambda t: (0, t, 0)),
            pl.BlockSpec((NBLK, TQ, 2 * CHUNK), lambda t: (0, t, 0)),
        ],
        out_specs=pl.BlockSpec((TQ, TK), lambda t: (t, 0)),
        out_shape=jax.ShapeDtypeStruct((QN, TK), jnp.int32),
    )(cval3, cidx3)


_NC = 2
_NS = 16
_NW = _NC * _NS


def _sc_gather_feats(features, idx2):
    mesh = plsc.VectorSubcoreMesh(core_axis_name="c", subcore_axis_name="s")

    @functools.partial(
        pl.kernel, mesh=mesh,
        out_type=jax.ShapeDtypeStruct((QN * TK, DN), jnp.float32),
        scratch_types=[
            pltpu.VMEM((2, 128), jnp.int32),
            pltpu.VMEM((128, DN), jnp.float32),
            pltpu.SemaphoreType.DMA,
        ],
    )
    def k(f_hbm, idx_hbm, out_hbm, idx_v, rows_v, sem):
        wid = lax.axis_index("s") * _NC + lax.axis_index("c")
        pltpu.sync_copy(idx_hbm.at[pl.ds(wid * 2, 2)], idx_v)
        for j in range(2):
            pltpu.async_copy(f_hbm.at[idx_v.at[j]], rows_v, sem).wait()
            pltpu.sync_copy(rows_v,
                            out_hbm.at[pl.ds(wid * 256 + j * 128, 128)])

    return k(features, idx2)


def kernel(queries, features):
    cval3, cidx3 = _matmul_top2(queries, features)
    fidx = _final_topk(cval3, cidx3)
    sel = _sc_gather_feats(features, fidx.reshape(64, 128))
    subgraph_feats = jnp.concatenate([queries, sel], axis=0)
    src = jnp.repeat(jnp.arange(QN, dtype=jnp.int64), TK)
    dst = QN + jnp.tile(jnp.arange(TK, dtype=jnp.int64), QN)
    edge_index = jnp.stack([src, dst], axis=0)
    return subgraph_feats, edge_index
